# Initial kernel scaffold; baseline (speedup 1.0000x reference)
import jax
import jax.numpy as jnp
from jax.experimental import pallas as pl


def kernel(path, free, collided, obstacles, edge_index, loop, node_w1, node_b1, bn_g, bn_b, node_w2, node_b2, l0a_w, l0a_b, l0b_w, l0b_b, l1a_w, l1a_b, l1b_w, l1b_b, sm_w, sm_b):
    raise NotImplementedError("write your pallas kernel here")


# speedup vs baseline: 1.0726x; 1.0726x over previous
import functools
import jax
import jax.numpy as jnp
from jax import lax
from jax.experimental import pallas as pl
from jax.experimental.pallas import tpu as pltpu

_TR = 2000


import numpy as np

_I0 = np.int32(0)


def _stage1_body(xn_ref, w1_ref, b1_ref, h_ref, s1_ref, s2_ref):
    h = jnp.dot(xn_ref[...], w1_ref[...], preferred_element_type=jnp.float32)
    h = h + b1_ref[...]
    h_ref[...] = h
    s1_ref[...] = jnp.sum(h, axis=0, keepdims=True)[None]
    s2_ref[...] = jnp.sum(h * h, axis=0, keepdims=True)[None]


def _node_stage1(xn, w1, b1):
    n = xn.shape[0]
    g = n // _TR
    kf = pl.pallas_call(
        _stage1_body,
        grid=(g,),
        in_specs=[
            pl.BlockSpec((_TR, xn.shape[1]), lambda i: (i, _I0)),
            pl.BlockSpec((xn.shape[1], 64), lambda i: (_I0, _I0)),
            pl.BlockSpec((1, 64), lambda i: (_I0, _I0)),
        ],
        out_specs=[
            pl.BlockSpec((_TR, 64), lambda i: (i, _I0)),
            pl.BlockSpec((1, 1, 64), lambda i: (i, _I0, _I0)),
            pl.BlockSpec((1, 1, 64), lambda i: (i, _I0, _I0)),
        ],
        out_shape=[
            jax.ShapeDtypeStruct((n, 64), jnp.float32),
            jax.ShapeDtypeStruct((g, 1, 64), jnp.float32),
            jax.ShapeDtypeStruct((g, 1, 64), jnp.float32),
        ],
    )
    return kf(xn, w1, b1.reshape(1, 64))


def _stage2_body(h_ref, sc_ref, tt_ref, w2_ref, b2_ref, wa_ref, x_ref, a_ref):
    hb = h_ref[...] * sc_ref[...] + tt_ref[...]
    x = jnp.dot(jnp.maximum(hb, 0.0), w2_ref[...],
                preferred_element_type=jnp.float32) + b2_ref[...]
    x_ref[...] = x
    a_ref[...] = jnp.dot(x, wa_ref[...], preferred_element_type=jnp.float32)


def _node_stage2(h, sc, tt, w2, b2, wa):
    n = h.shape[0]
    g = n // _TR
    kf = pl.pallas_call(
        _stage2_body,
        grid=(g,),
        in_specs=[
            pl.BlockSpec((_TR, 64), lambda i: (i, _I0)),
            pl.BlockSpec((1, 64), lambda i: (_I0, _I0)),
            pl.BlockSpec((1, 64), lambda i: (_I0, _I0)),
            pl.BlockSpec((64, 64), lambda i: (_I0, _I0)),
            pl.BlockSpec((1, 64), lambda i: (_I0, _I0)),
            pl.BlockSpec((64, 64), lambda i: (_I0, _I0)),
        ],
        out_specs=[
            pl.BlockSpec((_TR, 64), lambda i: (i, _I0)),
            pl.BlockSpec((_TR, 64), lambda i: (i, _I0)),
        ],
        out_shape=[
            jax.ShapeDtypeStruct((n, 64), jnp.float32),
            jax.ShapeDtypeStruct((n, 64), jnp.float32),
        ],
    )
    return kf(h, sc.reshape(1, 64), tt.reshape(1, 64), w2, b2.reshape(1, 64), wa)


def kernel(path, free, collided, obstacles, edge_index, loop, node_w1, node_b1,
           bn_g, bn_b, node_w2, node_b2, l0a_w, l0a_b, l0b_w, l0b_b,
           l1a_w, l1a_b, l1b_w, l1b_b, sm_w, sm_b):
    P = path.shape[0]
    Fn = free.shape[0]
    C = collided.shape[0]
    N = P + Fn + C

    path = path.astype(jnp.float32)
    cand = jnp.concatenate([free, collided], axis=0).astype(jnp.float32)

    src0 = edge_index[0]
    dst0 = edge_index[1]
    sent = jnp.int64(N) * jnp.int64(N)
    keys = jnp.where(dst0 < P, dst0.astype(jnp.int64) * N + src0.astype(jnp.int64), sent)
    ks = jnp.sort(keys)
    valid = ks < sent
    first = jnp.concatenate([jnp.ones((1,), bool), ks[1:] != ks[:-1]])
    w_base = valid & first
    d_e = jnp.where(valid, (ks // N).astype(jnp.int32), 0)
    s_e = jnp.where(valid, (ks % N).astype(jnp.int32), 0)

    r = jnp.arange(N, dtype=jnp.int32)
    info = jnp.stack([(r < P).astype(jnp.float32),
                      ((r >= P) & (r < P + Fn)).astype(jnp.float32),
                      (r >= P + Fn).astype(jnp.float32)], axis=1)

    w0 = l0a_w[:64]
    w1m = l0a_w[64:128]
    w2m = l0a_w[128:]
    wa = w0 + w1m
    wb = w2m - w0

    c2 = jnp.sum(cand * cand, axis=1)

    for _ in range(2):
        nodes = jnp.concatenate([path, cand], axis=0)
        xn = jnp.concatenate([nodes, info], axis=1)
        h, s1, s2 = _node_stage1(xn, node_w1, node_b1)
        mu = jnp.sum(s1, axis=(0, 1)) / N
        var = jnp.sum(s2, axis=(0, 1)) / N - mu * mu
        sc = bn_g / jnp.sqrt(var + 1e-5)
        tt = bn_b - mu * sc
        x, a_tab = _node_stage2(h, sc, tt, node_w2, node_b2, wa)
        xp = x[:P]
        bb = xp @ wb + l0a_b

        p2 = jnp.sum(path * path, axis=1)
        d2 = p2[:, None] + c2[None, :] - 2.0 * (path @ cand.T)
        _, nn = lax.top_k(-d2, 10)

        nnrows = nn[d_e]
        match = jnp.any(nnrows == (s_e - P)[:, None], axis=1) & (s_e >= P)
        wgt = (w_base & ~match).astype(jnp.float32)

        relu_e = jnp.maximum(a_tab[s_e] + bb[d_e], 0.0) * wgt[:, None]
        aggu = jax.ops.segment_sum(relu_e, d_e, num_segments=P)
        cnt = jax.ops.segment_sum(wgt, d_e, num_segments=P) + 10.0

        relu_n = jnp.maximum(a_tab[nn + P] + bb[:, None, :], 0.0)
        aggu = aggu + jnp.sum(relu_n, axis=1)

        agg = aggu @ l0b_w + cnt[:, None] * l0b_b
        hnp = xp + jnp.maximum(agg @ l1a_w + l1a_b, 0.0) @ l1b_w + l1b_b
        sm = hnp @ sm_w + sm_b
        path = path.at[1:-1].set(sm[1:-1])

    return path


# speedup vs baseline: 2.4197x; 2.2559x over previous
import functools
import jax
import jax.numpy as jnp
from jax import lax
from jax.experimental import pallas as pl
from jax.experimental.pallas import tpu as pltpu

_TR = 2000


import numpy as np

_I0 = np.int32(0)


def _stage1_body(xn_ref, w1_ref, b1_ref, h_ref, s1_ref, s2_ref):
    h = jnp.dot(xn_ref[...], w1_ref[...], preferred_element_type=jnp.float32)
    h = h + b1_ref[...]
    h_ref[...] = h
    s1_ref[...] = jnp.sum(h, axis=0, keepdims=True)[None]
    s2_ref[...] = jnp.sum(h * h, axis=0, keepdims=True)[None]


def _node_stage1(xn, w1, b1):
    n = xn.shape[0]
    g = n // _TR
    kf = pl.pallas_call(
        _stage1_body,
        grid=(g,),
        in_specs=[
            pl.BlockSpec((_TR, xn.shape[1]), lambda i: (i, _I0)),
            pl.BlockSpec((xn.shape[1], 64), lambda i: (_I0, _I0)),
            pl.BlockSpec((1, 64), lambda i: (_I0, _I0)),
        ],
        out_specs=[
            pl.BlockSpec((_TR, 64), lambda i: (i, _I0)),
            pl.BlockSpec((1, 1, 64), lambda i: (i, _I0, _I0)),
            pl.BlockSpec((1, 1, 64), lambda i: (i, _I0, _I0)),
        ],
        out_shape=[
            jax.ShapeDtypeStruct((n, 64), jnp.float32),
            jax.ShapeDtypeStruct((g, 1, 64), jnp.float32),
            jax.ShapeDtypeStruct((g, 1, 64), jnp.float32),
        ],
    )
    return kf(xn, w1, b1.reshape(1, 64))


def _stage2_body(h_ref, sc_ref, tt_ref, w2_ref, b2_ref, wa_ref, x_ref, a_ref):
    hb = h_ref[...] * sc_ref[...] + tt_ref[...]
    x = jnp.dot(jnp.maximum(hb, 0.0), w2_ref[...],
                preferred_element_type=jnp.float32) + b2_ref[...]
    x_ref[...] = x
    a_ref[...] = jnp.dot(x, wa_ref[...], preferred_element_type=jnp.float32)


def _node_stage2(h, sc, tt, w2, b2, wa):
    n = h.shape[0]
    g = n // _TR
    kf = pl.pallas_call(
        _stage2_body,
        grid=(g,),
        in_specs=[
            pl.BlockSpec((_TR, 64), lambda i: (i, _I0)),
            pl.BlockSpec((1, 64), lambda i: (_I0, _I0)),
            pl.BlockSpec((1, 64), lambda i: (_I0, _I0)),
            pl.BlockSpec((64, 64), lambda i: (_I0, _I0)),
            pl.BlockSpec((1, 64), lambda i: (_I0, _I0)),
            pl.BlockSpec((64, 64), lambda i: (_I0, _I0)),
        ],
        out_specs=[
            pl.BlockSpec((_TR, 64), lambda i: (i, _I0)),
            pl.BlockSpec((_TR, 64), lambda i: (i, _I0)),
        ],
        out_shape=[
            jax.ShapeDtypeStruct((n, 64), jnp.float32),
            jax.ShapeDtypeStruct((n, 64), jnp.float32),
        ],
    )
    return kf(h, sc.reshape(1, 64), tt.reshape(1, 64), w2, b2.reshape(1, 64), wa)


def kernel(path, free, collided, obstacles, edge_index, loop, node_w1, node_b1,
           bn_g, bn_b, node_w2, node_b2, l0a_w, l0a_b, l0b_w, l0b_b,
           l1a_w, l1a_b, l1b_w, l1b_b, sm_w, sm_b):
    P = path.shape[0]
    Fn = free.shape[0]
    C = collided.shape[0]
    N = P + Fn + C

    path = path.astype(jnp.float32)
    cand = jnp.concatenate([free, collided], axis=0).astype(jnp.float32)

    src0 = edge_index[0]
    dst0 = edge_index[1]
    sent64 = jnp.int64(2**31 - 1)
    keys = jnp.where(dst0 < P, dst0.astype(jnp.int64) * N + src0.astype(jnp.int64), sent64)
    ks = jnp.sort(keys.astype(jnp.int32))
    sent = jnp.int32(2**31 - 1)
    m_rel = jnp.sum((ks < sent).astype(jnp.int32))
    cap = 65536

    r = jnp.arange(N, dtype=jnp.int32)
    info = jnp.stack([(r < P).astype(jnp.float32),
                      ((r >= P) & (r < P + Fn)).astype(jnp.float32),
                      (r >= P + Fn).astype(jnp.float32)], axis=1)

    w0 = l0a_w[:64]
    w1m = l0a_w[64:128]
    w2m = l0a_w[128:]
    wa = w0 + w1m
    wb = w2m - w0

    c2 = jnp.sum(cand * cand, axis=1)

    for _ in range(2):
        nodes = jnp.concatenate([path, cand], axis=0)
        xn = jnp.concatenate([nodes, info], axis=1)
        h, s1, s2 = _node_stage1(xn, node_w1, node_b1)
        mu = jnp.sum(s1, axis=(0, 1)) / N
        var = jnp.sum(s2, axis=(0, 1)) / N - mu * mu
        sc = bn_g / jnp.sqrt(var + 1e-5)
        tt = bn_b - mu * sc
        x, a_tab = _node_stage2(h, sc, tt, node_w2, node_b2, wa)
        xp = x[:P]
        bb = xp @ wb + l0a_b

        p2 = jnp.sum(path * path, axis=1)
        d2 = p2[:, None] + c2[None, :] - 2.0 * (path @ cand.T)
        _, nn = lax.top_k(-d2, 10)

        def base_agg(ks_slice):
            valid = ks_slice < sent
            first = jnp.concatenate(
                [valid[:1], (ks_slice[1:] != ks_slice[:-1]) & valid[1:]])
            d_e = jnp.where(valid, ks_slice // N, 0)
            s_e = jnp.where(valid, ks_slice % N, 0)
            nnrows = nn[d_e]
            match = jnp.any(nnrows == (s_e - P)[:, None], axis=1) & (s_e >= P)
            wgt = (first & ~match).astype(jnp.float32)
            relu_e = jnp.maximum(a_tab[s_e] + bb[d_e], 0.0) * wgt[:, None]
            aggu = jax.ops.segment_sum(relu_e, d_e, num_segments=P)
            cnt = jax.ops.segment_sum(wgt, d_e, num_segments=P)
            return aggu, cnt

        aggu, cnt = lax.cond(m_rel <= cap,
                             lambda: base_agg(ks[:cap]),
                             lambda: base_agg(ks))
        cnt = cnt + 10.0

        relu_n = jnp.maximum(a_tab[nn + P] + bb[:, None, :], 0.0)
        aggu = aggu + jnp.sum(relu_n, axis=1)

        agg = aggu @ l0b_w + cnt[:, None] * l0b_b
        hnp = xp + jnp.maximum(agg @ l1a_w + l1a_b, 0.0) @ l1b_w + l1b_b
        sm = hnp @ sm_w + sm_b
        path = path.at[1:-1].set(sm[1:-1])

    return path
